# baseline (speedup 1.0000x reference)
import jax
import jax.numpy as jnp
from jax.experimental import pallas as pl
from jax.experimental.pallas import tpu as pltpu

DICT_LEN = 9
NUM_LS = 8
DIM = 1024
SEQ = 2048
BZ = 16
BLK_S = 128


def _mapper_kernel(perm_ref, starts_ref, x_ref, w_ref, b_ref, o_ref, xall, sem):
    def in_copy(c):
        return pltpu.make_async_copy(
            x_ref.at[:, perm_ref[c], :], xall.at[c], sem.at[c]
        )

    def out_copy(c):
        return pltpu.make_async_copy(
            xall.at[c], o_ref.at[:, perm_ref[c], :], sem.at[c]
        )

    for c in range(BZ):
        in_copy(c).start()

    for e in range(NUM_LS):
        def body(c, carry):
            in_copy(c).wait()
            xj = xall[c].astype(jnp.bfloat16)
            yj = jax.lax.dot_general(
                xj, w_ref[e],
                dimension_numbers=(((1,), (0,)), ((), ())),
                preferred_element_type=jnp.float32,
            )
            xall[c] = yj + b_ref[e]
            out_copy(c).start()
            return carry

        jax.lax.fori_loop(starts_ref[e], starts_ref[e + 1], body, 0)

    for c in range(BZ):
        out_copy(c).wait()


def kernel(x, lang_ids, W, b):
    idx = jnp.clip(DICT_LEN - 2 - lang_ids, 0, NUM_LS - 1).astype(jnp.int32)
    perm = jnp.argsort(idx).astype(jnp.int32)
    starts = jnp.searchsorted(
        idx[perm], jnp.arange(NUM_LS + 1, dtype=jnp.int32)
    ).astype(jnp.int32)
    Wt = jnp.swapaxes(W, 1, 2).astype(jnp.bfloat16)
    grid = (SEQ // BLK_S,)
    out = pl.pallas_call(
        _mapper_kernel,
        grid_spec=pltpu.PrefetchScalarGridSpec(
            num_scalar_prefetch=2,
            grid=grid,
            in_specs=[
                pl.BlockSpec((BLK_S, BZ, DIM), lambda s, p, st: (s, 0, 0)),
                pl.BlockSpec((NUM_LS, DIM, DIM), lambda s, p, st: (0, 0, 0)),
                pl.BlockSpec((NUM_LS, DIM), lambda s, p, st: (0, 0)),
            ],
            out_specs=pl.BlockSpec((BLK_S, BZ, DIM), lambda s, p, st: (s, 0, 0)),
            scratch_shapes=[
                pltpu.VMEM((BZ, BLK_S, DIM), jnp.float32),
                pltpu.SemaphoreType.DMA((BZ,)),
            ],
        ),
        out_shape=jax.ShapeDtypeStruct((SEQ, BZ, DIM), jnp.float32),
    )(perm, starts, x, Wt, b)
    return out
